# baseline (speedup 1.0000x reference)
import functools

import jax
import jax.numpy as jnp
from jax import lax
from jax.experimental import pallas as pl
from jax.experimental.pallas import tpu as pltpu
from jax.experimental.pallas import tpu_sc as plsc

_VOCAB = 1000000
_BATCH = 4096
_HIST = 200
_D = 32
_B = _BATCH * _HIST
_LANES = 16


def _make_sc_gather():
    info = plsc.get_sparse_core_info()
    nc, ns = info.num_cores, info.num_subcores
    nw = nc * ns
    bb_blk = _BATCH // nw
    b_per_w = _B // nw
    hh = 10
    chunk = hh * bb_blk
    n_chunks = _HIST // hh
    mesh = plsc.VectorSubcoreMesh(core_axis_name="c", subcore_axis_name="s")

    @functools.partial(
        pl.kernel,
        out_type=jax.ShapeDtypeStruct((_B, _D), jnp.float32),
        mesh=mesh,
        scratch_types=[
            pltpu.VMEM((b_per_w,), jnp.int32),
            pltpu.VMEM((chunk,), jnp.int32),
            pltpu.VMEM((chunk,), jnp.int32),
            pltpu.VMEM((chunk, _D), jnp.float32),
            pltpu.VMEM((chunk, _D), jnp.float32),
            pltpu.SemaphoreType.DMA,
            pltpu.SemaphoreType.DMA,
            pltpu.SemaphoreType.DMA,
            pltpu.SemaphoreType.DMA,
        ],
        compiler_params=pltpu.CompilerParams(use_tc_tiling_on_sc=False,
                                             needs_layout_passes=False),
    )
    def gather_kernel(idx_hbm, table_hbm, out_hbm, idx_all, ih0, ih1,
                      rows0, rows1, gsem0, gsem1, wsem0, wsem1):
        wid = lax.axis_index("s") * nc + lax.axis_index("c")
        base = wid * b_per_w
        pltpu.sync_copy(idx_hbm.at[pl.ds(pl.multiple_of(base, 8), b_per_w)],
                        idx_all)

        iota = lax.iota(jnp.int32, _LANES)
        perm_static = [iota * _HIST + bb0 * _LANES * _HIST
                       for bb0 in range(bb_blk // _LANES)]

        ihs = [ih0, ih1]
        rows = [rows0, rows1]
        gsems = [gsem0, gsem1]
        wsems = [wsem0, wsem1]
        gcopy = [None, None]
        wcopy = [None, None]

        def build_idx(g, b):
            for hl in range(hh):
                h = g * hh + hl
                for bb0 in range(bb_blk // _LANES):
                    vals = plsc.load_gather(idx_all, [perm_static[bb0] + h])
                    ihs[b][pl.ds(hl * bb_blk + bb0 * _LANES, _LANES)] = vals

        def start_gather(g, b):
            build_idx(g, b)
            gcopy[b] = pltpu.async_copy(table_hbm.at[ihs[b]], rows[b],
                                        gsems[b])

        start_gather(0, 0)
        for g in range(n_chunks):
            b = g % 2
            if g + 1 < n_chunks:
                if g >= 1:
                    for w in wcopy[b ^ 1]:
                        w.wait()
                start_gather(g + 1, b ^ 1)
            gcopy[b].wait()
            wcopy[b] = [
                pltpu.async_copy(
                    rows[b].at[pl.ds(hl * bb_blk, bb_blk)],
                    out_hbm.at[pl.ds(
                        (g * hh + hl) * _BATCH + wid * bb_blk, bb_blk)],
                    wsems[b])
                for hl in range(hh)
            ]
        for ws in wcopy:
            for w in ws:
                w.wait()

    return gather_kernel


def _make_relayout():
    info = plsc.get_sparse_core_info()
    nc, ns = info.num_cores, info.num_subcores
    nw = nc * ns
    bb_blk = _BATCH // nw
    hh = 10
    n_win = _HIST // hh
    win_words = bb_blk * hh * _D
    mesh = plsc.VectorSubcoreMesh(core_axis_name="c", subcore_axis_name="s")

    @functools.partial(
        pl.kernel,
        out_type=jax.ShapeDtypeStruct((_HIST, _D, _BATCH), jnp.float32),
        mesh=mesh,
        scratch_types=[
            pltpu.VMEM((win_words,), jnp.float32),
            pltpu.VMEM((win_words,), jnp.float32),
        ] + [pltpu.VMEM((8, 128), jnp.float32)] * 8 + [
            pltpu.SemaphoreType.DMA,
            pltpu.SemaphoreType.DMA,
        ] + [pltpu.SemaphoreType.DMA] * 8,
        compiler_params=pltpu.CompilerParams(use_tc_tiling_on_sc=True,
                                             needs_layout_passes=False),
    )
    def relayout_kernel(rows_hbm, out_hbm, win0, win1, t0, t1, t2, t3, t4,
                        t5, t6, t7, rsem0, rsem1, w0, w1, w2, w3, w4, w5,
                        w6, w7):
        wid = lax.axis_index("s") * nc + lax.axis_index("c")

        wins = [win0, win1]
        rsems = [rsem0, rsem1]
        tiles = [t0, t1, t2, t3, t4, t5, t6, t7]
        wsems = [w0, w1, w2, w3, w4, w5, w6, w7]
        rcopy = [None, None]

        iota = lax.iota(jnp.int32, _LANES)
        src_static = [iota * _D + bb0 * _LANES * _D for bb0 in range(8)]

        def start_read(v, b):
            rcopy[b] = [
                pltpu.async_copy(
                    rows_hbm.at[pl.ds(
                        pl.multiple_of(
                            ((v * hh + hl) * _BATCH + wid * bb_blk) * _D, 8),
                        bb_blk * _D)],
                    wins[b].at[pl.ds(hl * bb_blk * _D, bb_blk * _D)],
                    rsems[b])
                for hl in range(hh)
            ]

        def drain_tile(t):
            pltpu.make_async_copy(
                tiles[t], out_hbm.at[0, pl.ds(0, 8), pl.ds(0, 128)],
                wsems[t]).wait()

        def do_h(v, b, hl, parity):
            h = v * hh + hl
            for kg in range(4):
                t = parity * 4 + kg
                @pl.when(h >= 2)
                def _():
                    drain_tile(t)
                for kk in range(8):
                    base_kk = hl * (bb_blk * _D) + kg * 8 + kk
                    for bb0 in range(8):
                        vals = plsc.load_gather(
                            wins[b], [src_static[bb0] + base_kk])
                        tiles[t][kk, pl.ds(bb0 * _LANES, _LANES)] = vals
                pltpu.async_copy(
                    tiles[t],
                    out_hbm.at[h, pl.ds(kg * 8, 8),
                               pl.ds(wid * bb_blk, bb_blk)],
                    wsems[t])

        def process_window(v, b):
            for c in rcopy[b]:
                c.wait()

            @pl.loop(0, hh, step=2)
            def _(hl):
                do_h(v, b, hl, 0)
                do_h(v, b, hl + 1, 1)

        start_read(0, 0)

        @pl.loop(0, n_win, step=2)
        def _(v):
            start_read(v + 1, 1)
            process_window(v, 0)

            @pl.when(v + 2 < n_win)
            def _():
                start_read(v + 2, 0)

            process_window(v + 1, 1)

        for t in range(8):
            drain_tile(t)

    return relayout_kernel


@jax.jit
def kernel(x, table):
    idx = x.astype(jnp.int32).reshape(_B)
    rows = _make_sc_gather()(idx, table)
    out_t = _make_relayout()(rows.reshape(_B * _D))
    return jnp.transpose(out_t, (2, 0, 1))


# speedup vs baseline: 1.1317x; 1.1317x over previous
import functools

import jax
import jax.numpy as jnp
from jax import lax
from jax.experimental import pallas as pl
from jax.experimental.pallas import tpu as pltpu
from jax.experimental.pallas import tpu_sc as plsc

_VOCAB = 1000000
_BATCH = 4096
_HIST = 200
_D = 32
_B = _BATCH * _HIST
_LANES = 16


def _make_sc_gather():
    info = plsc.get_sparse_core_info()
    nc, ns = info.num_cores, info.num_subcores
    nw = nc * ns
    bb_blk = _BATCH // nw
    b_per_w = _B // nw
    hh = 10
    chunk = hh * bb_blk
    n_chunks = _HIST // hh
    mesh = plsc.VectorSubcoreMesh(core_axis_name="c", subcore_axis_name="s")

    @functools.partial(
        pl.kernel,
        out_type=jax.ShapeDtypeStruct((_B, _D), jnp.float32),
        mesh=mesh,
        scratch_types=[
            pltpu.VMEM((b_per_w,), jnp.int32),
            pltpu.VMEM((chunk,), jnp.int32),
            pltpu.VMEM((chunk,), jnp.int32),
            pltpu.VMEM((chunk, _D), jnp.float32),
            pltpu.VMEM((chunk, _D), jnp.float32),
            pltpu.SemaphoreType.DMA,
            pltpu.SemaphoreType.DMA,
            pltpu.SemaphoreType.DMA,
            pltpu.SemaphoreType.DMA,
        ],
        compiler_params=pltpu.CompilerParams(use_tc_tiling_on_sc=False,
                                             needs_layout_passes=False),
    )
    def gather_kernel(idx_hbm, table_hbm, out_hbm, idx_all, ih0, ih1,
                      rows0, rows1, gsem0, gsem1, wsem0, wsem1):
        wid = lax.axis_index("s") * nc + lax.axis_index("c")
        base = wid * b_per_w
        pltpu.sync_copy(idx_hbm.at[pl.ds(pl.multiple_of(base, 8), b_per_w)],
                        idx_all)

        iota = lax.iota(jnp.int32, _LANES)
        perm_static = [iota * _HIST + bb0 * _LANES * _HIST
                       for bb0 in range(bb_blk // _LANES)]

        ihs = [ih0, ih1]
        rows = [rows0, rows1]
        gsems = [gsem0, gsem1]
        wsems = [wsem0, wsem1]
        gcopy = [None, None]
        wcopy = [None, None]

        def build_idx(g, b):
            for hl in range(hh):
                h = g * hh + hl
                for bb0 in range(bb_blk // _LANES):
                    vals = plsc.load_gather(idx_all, [perm_static[bb0] + h])
                    ihs[b][pl.ds(hl * bb_blk + bb0 * _LANES, _LANES)] = vals

        def start_gather(g, b):
            build_idx(g, b)
            gcopy[b] = pltpu.async_copy(table_hbm.at[ihs[b]], rows[b],
                                        gsems[b])

        start_gather(0, 0)
        for g in range(n_chunks):
            b = g % 2
            if g + 1 < n_chunks:
                if g >= 1:
                    for w in wcopy[b ^ 1]:
                        w.wait()
                start_gather(g + 1, b ^ 1)
            gcopy[b].wait()
            wcopy[b] = [
                pltpu.async_copy(
                    rows[b].at[pl.ds(hl * bb_blk, bb_blk)],
                    out_hbm.at[pl.ds(
                        (g * hh + hl) * _BATCH + wid * bb_blk, bb_blk)],
                    wsems[b])
                for hl in range(hh)
            ]
        for ws in wcopy:
            for w in ws:
                w.wait()

    return gather_kernel


def _make_relayout():
    info = plsc.get_sparse_core_info()
    nc, ns = info.num_cores, info.num_subcores
    nw = nc * ns
    bb_blk = _BATCH // nw
    hh = 10
    n_win = _HIST // hh
    win_words = bb_blk * hh * _D
    mesh = plsc.VectorSubcoreMesh(core_axis_name="c", subcore_axis_name="s")

    @functools.partial(
        pl.kernel,
        out_type=jax.ShapeDtypeStruct((_HIST, _D, _BATCH), jnp.float32),
        mesh=mesh,
        scratch_types=[
            pltpu.VMEM((win_words,), jnp.float32),
            pltpu.VMEM((win_words,), jnp.float32),
        ] + [pltpu.VMEM((8, 128), jnp.float32)] * 8 + [
            pltpu.SemaphoreType.DMA,
            pltpu.SemaphoreType.DMA,
        ] + [pltpu.SemaphoreType.DMA] * 8,
        compiler_params=pltpu.CompilerParams(use_tc_tiling_on_sc=True,
                                             needs_layout_passes=False),
    )
    def relayout_kernel(rows_hbm, out_hbm, win0, win1, t0, t1, t2, t3, t4,
                        t5, t6, t7, rsem0, rsem1, w0, w1, w2, w3, w4, w5,
                        w6, w7):
        wid = lax.axis_index("s") * nc + lax.axis_index("c")

        wins = [win0, win1]
        rsems = [rsem0, rsem1]
        tiles = [t0, t1, t2, t3, t4, t5, t6, t7]
        wsems = [w0, w1, w2, w3, w4, w5, w6, w7]
        rcopy = [None, None]

        iota = lax.iota(jnp.int32, _LANES)
        src_static = [iota * _D + bb0 * _LANES * _D for bb0 in range(8)]

        def start_read(v, b):
            rcopy[b] = [
                pltpu.async_copy(
                    rows_hbm.at[pl.ds(
                        pl.multiple_of(
                            ((v * hh + hl) * _BATCH + wid * bb_blk) * _D, 8),
                        bb_blk * _D)],
                    wins[b].at[pl.ds(hl * bb_blk * _D, bb_blk * _D)],
                    rsems[b])
                for hl in range(hh)
            ]

        def drain_tile(t):
            pltpu.make_async_copy(
                tiles[t], out_hbm.at[0, pl.ds(0, 8), pl.ds(0, 128)],
                wsems[t]).wait()

        def do_h(v, b, hl, parity):
            h = v * hh + hl
            for kg in range(4):
                t = parity * 4 + kg
                @pl.when(h >= 2)
                def _():
                    drain_tile(t)
                for kk2 in range(4):
                    vals = []
                    for kk in (2 * kk2, 2 * kk2 + 1):
                        base_kk = hl * (bb_blk * _D) + kg * 8 + kk
                        vals.extend(
                            plsc.load_gather(wins[b],
                                             [src_static[bb0] + base_kk])
                            for bb0 in range(8))
                    i = 0
                    for kk in (2 * kk2, 2 * kk2 + 1):
                        for bb0 in range(8):
                            tiles[t][kk, pl.ds(bb0 * _LANES, _LANES)] = vals[i]
                            i += 1
                pltpu.async_copy(
                    tiles[t],
                    out_hbm.at[h, pl.ds(kg * 8, 8),
                               pl.ds(wid * bb_blk, bb_blk)],
                    wsems[t])

        def process_window(v, b):
            for c in rcopy[b]:
                c.wait()

            @pl.loop(0, hh, step=2)
            def _(hl):
                do_h(v, b, hl, 0)
                do_h(v, b, hl + 1, 1)

        start_read(0, 0)

        @pl.loop(0, n_win, step=2)
        def _(v):
            start_read(v + 1, 1)
            process_window(v, 0)

            @pl.when(v + 2 < n_win)
            def _():
                start_read(v + 2, 0)

            process_window(v + 1, 1)

        for t in range(8):
            drain_tile(t)

    return relayout_kernel


@jax.jit
def kernel(x, table):
    idx = x.astype(jnp.int32).reshape(_B)
    rows = _make_sc_gather()(idx, table)
    out_t = _make_relayout()(rows.reshape(_B * _D))
    return jnp.transpose(out_t, (2, 0, 1))
